# Initial kernel scaffold; baseline (speedup 1.0000x reference)
import jax
import jax.numpy as jnp
from jax.experimental import pallas as pl


def kernel(x, w1, b1, w2, b2):
    raise NotImplementedError("write your pallas kernel here")


# speedup vs baseline: 1.4978x; 1.4978x over previous
import jax
import jax.numpy as jnp
from jax.experimental import pallas as pl
from jax.experimental.pallas import tpu as pltpu

IN = 4
HID = 10
OUT = 3
GROUP = 32
HID_L = 12
LANE = GROUP * IN
H_LANES = GROUP * HID_L
O_LANES = GROUP * OUT
BLOCK_R = 512


def _mlp_body(x_ref, w1_ref, b1_ref, w2_ref, b2_ref, o_ref):
    h = jnp.dot(x_ref[...], w1_ref[...],
                preferred_element_type=jnp.float32)
    h = jax.nn.sigmoid(h + b1_ref[...])
    y = jnp.dot(h, w2_ref[...],
                preferred_element_type=jnp.float32)
    o_ref[...] = jax.nn.sigmoid(y + b2_ref[...])


def _pack_params(w1, b1, w2, b2):
    eye = jnp.eye(GROUP, dtype=jnp.float32)
    w1p = jnp.zeros((IN, HID_L), jnp.float32).at[:, :HID].set(w1)
    w2p = jnp.zeros((HID_L, OUT), jnp.float32).at[:HID, :].set(w2)
    w1b = jnp.kron(eye, w1p)
    w2b = jnp.kron(eye, w2p)
    b1b = jnp.tile(jnp.pad(b1, (0, HID_L - HID)), GROUP).reshape(1, H_LANES)
    b2b = jnp.tile(b2, GROUP).reshape(1, O_LANES)
    return w1b, b1b, w2b, b2b


def kernel(x, w1, b1, w2, b2):
    B = x.shape[0]
    x = x.astype(jnp.float32)
    w1b, b1b, w2b, b2b = _pack_params(w1.astype(jnp.float32),
                                      b1.astype(jnp.float32),
                                      w2.astype(jnp.float32),
                                      b2.astype(jnp.float32))

    Bp = -(-B // GROUP) * GROUP
    if Bp != B:
        x = jnp.pad(x, ((0, Bp - B), (0, 0)))
    R = Bp // GROUP
    xv = x.reshape(R, LANE)

    br = min(BLOCK_R, -(-R // 8) * 8)
    out = pl.pallas_call(
        _mlp_body,
        out_shape=jax.ShapeDtypeStruct((R, O_LANES), jnp.float32),
        grid=(pl.cdiv(R, br),),
        in_specs=[
            pl.BlockSpec((br, LANE), lambda i: (i, 0)),
            pl.BlockSpec((LANE, H_LANES), lambda i: (0, 0)),
            pl.BlockSpec((1, H_LANES), lambda i: (0, 0)),
            pl.BlockSpec((H_LANES, O_LANES), lambda i: (0, 0)),
            pl.BlockSpec((1, O_LANES), lambda i: (0, 0)),
        ],
        out_specs=pl.BlockSpec((br, O_LANES), lambda i: (i, 0)),
        compiler_params=pltpu.CompilerParams(
            dimension_semantics=("parallel",),
            vmem_limit_bytes=64 * 1024 * 1024,
        ),
    )(xv, w1b, b1b, w2b, b2b)

    return out.reshape(-1)[: OUT * B]
